# baseline (speedup 1.0000x reference)
import functools

import jax
import jax.numpy as jnp
from jax import lax
from jax.experimental import pallas as pl
from jax.experimental.pallas import tpu as pltpu
from jax.experimental.pallas import tpu_sc as plsc

N = 10000
E = 320000
D = 128
BLOCKS = 6

NC = 2
NS = 16
NW = NC * NS
NPAD = 10240
ROWS_PER_SUB = NPAD // NS
CHUNK = 80
RING = 4
NCHUNKS = 128
EDGES_PER_W = NCHUNKS * CHUNK
EPAD = EDGES_PER_W * NW


def _segsum_body(z_hbm, src_hbm, dst_hbm, zeros_hbm, out0_hbm, out1_hbm,
                 idx_s, idx_d, rows, acc, *sems):
    c = lax.axis_index("c")
    s = lax.axis_index("s")
    w = s * NC + c
    gsems = sems[:RING]
    isems = sems[RING:]

    def idx_start(i, sl):
        pltpu.async_copy(src_hbm.at[w, i], idx_s.at[sl], isems[sl])
        pltpu.async_copy(dst_hbm.at[w, i], idx_d.at[sl], isems[sl])

    def idx_wait(i, sl):
        pltpu.make_async_copy(src_hbm.at[w, i], idx_s.at[sl],
                              isems[sl]).wait()
        pltpu.make_async_copy(dst_hbm.at[w, i], idx_d.at[sl],
                              isems[sl]).wait()

    def gather_start(sl, j):
        pltpu.async_copy(z_hbm.at[idx_s.at[sl]], rows.at[j], gsems[j])

    def gather_wait(sl, j):
        pltpu.make_async_copy(z_hbm.at[idx_s.at[sl]], rows.at[j],
                              gsems[j]).wait()

    def scatter_add(sl, j):
        pltpu.sync_copy(rows.at[j], acc.at[idx_d.at[sl]], add=True)

    for sl in range(2 * RING):
        idx_start(sl, sl)
    for j in range(RING):
        idx_wait(j, j)
        gather_start(j, j)

    for j in range(ROWS_PER_SUB // 128):
        pltpu.sync_copy(zeros_hbm,
                        acc.at[pl.ds(s * ROWS_PER_SUB + j * 128, 128)])
    plsc.subcore_barrier()

    def body(k, carry):
        for p in range(2):
            for j in range(RING):
                i = 2 * RING * k + RING * p + j
                sl = RING * p + j
                nsl = (sl + RING) % (2 * RING)
                gather_wait(sl, j)
                scatter_add(sl, j)

                @pl.when(i + 2 * RING < NCHUNKS)
                def _():
                    idx_start(i + 2 * RING, sl)

                @pl.when(i + RING < NCHUNKS)
                def _():
                    idx_wait(i + RING, nsl)
                    gather_start(nsl, j)
        return carry

    lax.fori_loop(0, NCHUNKS // (2 * RING), body, 0)

    plsc.subcore_barrier()

    sl = pl.ds(s * ROWS_PER_SUB, ROWS_PER_SUB)

    @pl.when(c == 0)
    def _():
        pltpu.sync_copy(acc.at[sl], out0_hbm.at[sl])

    @pl.when(c == 1)
    def _():
        pltpu.sync_copy(acc.at[sl], out1_hbm.at[sl])


@functools.partial(
    pl.kernel,
    mesh=plsc.VectorSubcoreMesh(core_axis_name="c", subcore_axis_name="s"),
    out_type=[
        jax.ShapeDtypeStruct((NPAD, D), jnp.float32),
        jax.ShapeDtypeStruct((NPAD, D), jnp.float32),
    ],
    scratch_types=[
        pltpu.VMEM((2 * RING, CHUNK), jnp.int32),
        pltpu.VMEM((2 * RING, CHUNK), jnp.int32),
        pltpu.VMEM((RING, CHUNK, D), jnp.float32),
        pltpu.VMEM_SHARED((NPAD, D), jnp.float32),
    ] + [pltpu.SemaphoreType.DMA] * (3 * RING),
)
def _segsum(z_hbm, src_hbm, dst_hbm, zeros_hbm, out0_hbm, out1_hbm, *scratch):
    _segsum_body(z_hbm, src_hbm, dst_hbm, zeros_hbm, out0_hbm, out1_hbm,
                 *scratch)


ROWS_BLK = 1000


def _combine_kernel(u0_ref, u1_ref, z_ref, w_ref, wl_ref, b_ref, out_ref):
    acc = jnp.dot(u0_ref[...] + u1_ref[...], w_ref[...],
                  preferred_element_type=jnp.float32)
    acc = acc + jnp.dot(z_ref[...], wl_ref[...],
                        preferred_element_type=jnp.float32)
    out_ref[...] = acc + b_ref[...]


def _combine_res_kernel(u0_ref, u1_ref, z_ref, w_ref, wl_ref, b_ref, h_ref,
                        out_ref):
    acc = jnp.dot(u0_ref[...] + u1_ref[...], w_ref[...],
                  preferred_element_type=jnp.float32)
    acc = acc + jnp.dot(z_ref[...], wl_ref[...],
                        preferred_element_type=jnp.float32)
    out_ref[...] = (h_ref[...] + acc + b_ref[...]) * 0.5


_row_spec = pl.BlockSpec((ROWS_BLK, D), lambda i: (i, 0))
_mat_spec = pl.BlockSpec((D, D), lambda i: (0, 0))
_bias_spec = pl.BlockSpec((1, D), lambda i: (0, 0))


def _combine(u0, u1, z, W, Wl, b):
    return pl.pallas_call(
        _combine_kernel,
        grid=(N // ROWS_BLK,),
        in_specs=[_row_spec, _row_spec, _row_spec, _mat_spec, _mat_spec,
                  _bias_spec],
        out_specs=_row_spec,
        out_shape=jax.ShapeDtypeStruct((N, D), jnp.float32),
    )(u0, u1, z, W, Wl, b.reshape(1, D))


def _combine_res(u0, u1, z, W, Wl, b, h):
    return pl.pallas_call(
        _combine_res_kernel,
        grid=(N // ROWS_BLK,),
        in_specs=[_row_spec, _row_spec, _row_spec, _mat_spec, _mat_spec,
                  _bias_spec, _row_spec],
        out_specs=_row_spec,
        out_shape=jax.ShapeDtypeStruct((N, D), jnp.float32),
    )(u0, u1, z, W, Wl, b.reshape(1, D), h)


def kernel(x, edge_index, W_in, Wl_in, b_in, blocks_W, blocks_Wl, blocks_b,
           W_out, Wl_out, b_out):
    src = jnp.concatenate(
        [edge_index[0], jnp.zeros((EPAD - E,), jnp.int32)]
    ).reshape(NW, NCHUNKS, CHUNK)
    dst = jnp.concatenate(
        [edge_index[1], jnp.full((EPAD - E,), NPAD - 1, jnp.int32)]
    ).reshape(NW, NCHUNKS, CHUNK)
    zeros = jnp.zeros((128, D), jnp.float32)

    def gconv(z, W, Wl, b):
        u0, u1 = _segsum(z, src, dst, zeros)
        return _combine(u0, u1, z, W, Wl, b)

    def gconv_res(z, W, Wl, b, h):
        u0, u1 = _segsum(z, src, dst, zeros)
        return _combine_res(u0, u1, z, W, Wl, b, h)

    h = gconv(x, W_in, Wl_in, b_in)
    for i in range(BLOCKS):
        t = gconv(h, blocks_W[i, 0], blocks_Wl[i, 0], blocks_b[i, 0])
        h = gconv_res(t, blocks_W[i, 1], blocks_Wl[i, 1], blocks_b[i, 1], h)
    x_out = gconv(h, W_out, Wl_out, b_out)
    return (x_out, h)
